# baseline (speedup 1.0000x reference)
import functools

import jax
import jax.numpy as jnp
from jax import lax
from jax.experimental import pallas as pl
from jax.experimental.pallas import tpu as pltpu
from jax.experimental.pallas import tpu_sc as plsc

MAX_ROWS = 8192
D = 1024
NB = 4
NT = 8192

NC = 2
NS = 16
NW = NC * NS

N = NB * NT
ROWS_PER_W = N // NW
K = 32
CHUNKS = ROWS_PER_W // K
LANES = 16
VECS_PER_ROW = D // LANES


def _sc_body(x_hbm, idx_hbm, table_hbm, out_hbm, idx_v, bufx, bufg, sem_x, sem_g):
    wid = lax.axis_index("s") * NC + lax.axis_index("c")
    row0 = wid * ROWS_PER_W

    pltpu.sync_copy(idx_hbm.at[pl.ds(row0, ROWS_PER_W)], idx_v)

    def chunk_body(c, carry):
        base = row0 + c * K
        cp_x = pltpu.async_copy(
            x_hbm.at[pl.ds(base * D, K * D)], bufx, sem_x)
        cp_g = pltpu.async_copy(
            table_hbm.at[idx_v.at[pl.ds(c * K, K)]], bufg, sem_g)
        cp_x.wait()
        cp_g.wait()

        def row_body(r, rc):
            for v in range(VECS_PER_ROW):
                off = r * D + v * LANES
                bufx[pl.ds(off, LANES)] = (
                    bufx[pl.ds(off, LANES)] + bufg[r, pl.ds(v * LANES, LANES)])
            return rc

        lax.fori_loop(0, K, row_body, 0, unroll=False)

        pltpu.sync_copy(bufx, out_hbm.at[pl.ds(base * D, K * D)])
        return carry

    lax.fori_loop(0, CHUNKS, chunk_body, 0, unroll=False)


@functools.partial(jax.jit, donate_argnums=())
def kernel(x, times_bt, embeddings_tc):
    xf = x.reshape(N * D)
    idx = times_bt.astype(jnp.int32).reshape(N)

    mesh = plsc.VectorSubcoreMesh(
        core_axis_name="c", subcore_axis_name="s", num_cores=NC,
        num_subcores=NS)
    run = pl.kernel(
        _sc_body,
        out_type=jax.ShapeDtypeStruct((N * D,), jnp.float32),
        mesh=mesh,
        scratch_types=[
            pltpu.VMEM((ROWS_PER_W,), jnp.int32),
            pltpu.VMEM((K * D,), jnp.float32),
            pltpu.VMEM((K, D), jnp.float32),
            pltpu.SemaphoreType.DMA,
            pltpu.SemaphoreType.DMA,
        ],
    )
    out = run(xf, idx, embeddings_tc)
    return out.reshape(NB, NT, D)


# speedup vs baseline: 1.7047x; 1.7047x over previous
import functools

import jax
import jax.numpy as jnp
from jax import lax
from jax.experimental import pallas as pl
from jax.experimental.pallas import tpu as pltpu
from jax.experimental.pallas import tpu_sc as plsc

MAX_ROWS = 8192
D = 1024
NB = 4
NT = 8192

NC = 2
NS = 16
NW = NC * NS

N = NB * NT
ROWS_PER_W = N // NW
K = 16
CHUNKS = ROWS_PER_W // K
LANES = 16
VECS_PER_ROW = D // LANES


def _sc_body(x_hbm, idx_hbm, table_hbm, out_hbm,
             idx_v, bufx0, bufx1, bufg0, bufg1, bufo0, bufo1,
             sx0, sx1, sg0, sg1, so0, so1):
    wid = lax.axis_index("s") * NC + lax.axis_index("c")
    row0 = wid * ROWS_PER_W

    bufxs = (bufx0, bufx1)
    bufgs = (bufg0, bufg1)
    bufos = (bufo0, bufo1)
    sxs = (sx0, sx1)
    sgs = (sg0, sg1)
    sos = (so0, so1)

    pltpu.sync_copy(idx_hbm.at[pl.ds(row0, ROWS_PER_W)], idx_v)

    def in_descs(c, b):
        base = row0 + c * K
        dx = pltpu.make_async_copy(
            x_hbm.at[pl.ds(base * D, K * D)], bufxs[b], sxs[b])
        dg = pltpu.make_async_copy(
            table_hbm.at[idx_v.at[pl.ds(c * K, K)]], bufgs[b], sgs[b])
        return dx, dg

    def out_desc(c, b):
        base = row0 + c * K
        return pltpu.make_async_copy(
            bufos[b], out_hbm.at[pl.ds(base * D, K * D)], sos[b])

    for c in range(2):
        dx, dg = in_descs(c, c)
        dx.start()
        dg.start()

    def two_chunks(cc, carry):
        for b in range(2):
            c = 2 * cc + b
            dx, dg = in_descs(c, b)
            dx.wait()
            dg.wait()

            @pl.when(c >= 2)
            def _wait_out():
                out_desc(c - 2, b).wait()

            bx, bg, bo = bufxs[b], bufgs[b], bufos[b]

            @plsc.parallel_loop(0, K, 1)
            def _add_row(r):
                for v in range(VECS_PER_ROW):
                    off = r * D + v * LANES
                    bo[pl.ds(off, LANES)] = (
                        bx[pl.ds(off, LANES)] + bg[r, pl.ds(v * LANES, LANES)])

            out_desc(c, b).start()

            @pl.when(c + 2 < CHUNKS)
            def _issue_next():
                ndx, ndg = in_descs(c + 2, b)
                ndx.start()
                ndg.start()
        return carry

    lax.fori_loop(0, CHUNKS // 2, two_chunks, 0, unroll=False)

    out_desc(CHUNKS - 2, 0).wait()
    out_desc(CHUNKS - 1, 1).wait()


@functools.partial(jax.jit, donate_argnums=())
def kernel(x, times_bt, embeddings_tc):
    xf = x.reshape(N * D)
    idx = times_bt.astype(jnp.int32).reshape(N)

    mesh = plsc.VectorSubcoreMesh(
        core_axis_name="c", subcore_axis_name="s", num_cores=NC,
        num_subcores=NS)
    run = pl.kernel(
        _sc_body,
        out_type=jax.ShapeDtypeStruct((N * D,), jnp.float32),
        mesh=mesh,
        scratch_types=[
            pltpu.VMEM((ROWS_PER_W,), jnp.int32),
            pltpu.VMEM((K * D,), jnp.float32),
            pltpu.VMEM((K * D,), jnp.float32),
            pltpu.VMEM((K, D), jnp.float32),
            pltpu.VMEM((K, D), jnp.float32),
            pltpu.VMEM((K * D,), jnp.float32),
            pltpu.VMEM((K * D,), jnp.float32),
            pltpu.SemaphoreType.DMA,
            pltpu.SemaphoreType.DMA,
            pltpu.SemaphoreType.DMA,
            pltpu.SemaphoreType.DMA,
            pltpu.SemaphoreType.DMA,
            pltpu.SemaphoreType.DMA,
        ],
    )
    out = run(xf, idx, embeddings_tc)
    return out.reshape(NB, NT, D)


# speedup vs baseline: 4.3468x; 2.5499x over previous
import functools

import jax
import jax.numpy as jnp
from jax import lax
from jax.experimental import pallas as pl
from jax.experimental.pallas import tpu as pltpu
from jax.experimental.pallas import tpu_sc as plsc

MAX_ROWS = 8192
D = 1024
NB = 4
NT = 8192

NC = 2
NS = 16
NW = NC * NS

N = NB * NT
ROWS_PER_W = N // NW
K = 16
CHUNKS = ROWS_PER_W // K
LANES = 16
VECS_PER_ROW = D // LANES


def _sc_body(x_hbm, idx_hbm, table_hbm, out_hbm,
             idx_v, bufx0, bufx1, bufg0, bufg1, bufo0, bufo1,
             sx0, sx1, sg0, sg1, so0, so1):
    wid = lax.axis_index("s") * NC + lax.axis_index("c")
    row0 = wid * ROWS_PER_W

    bufxs = (bufx0, bufx1)
    bufgs = (bufg0, bufg1)
    bufos = (bufo0, bufo1)
    sxs = (sx0, sx1)
    sgs = (sg0, sg1)
    sos = (so0, so1)

    pltpu.sync_copy(idx_hbm.at[pl.ds(row0, ROWS_PER_W)], idx_v)

    def in_descs(c, b):
        base = row0 + c * K
        dx = pltpu.make_async_copy(
            x_hbm.at[pl.ds(base, K), :], bufxs[b], sxs[b])
        dg = pltpu.make_async_copy(
            table_hbm.at[idx_v.at[pl.ds(c * K, K)]], bufgs[b], sgs[b])
        return dx, dg

    def out_desc(c, b):
        base = row0 + c * K
        return pltpu.make_async_copy(
            bufos[b], out_hbm.at[pl.ds(base, K), :], sos[b])

    for c in range(2):
        dx, dg = in_descs(c, c)
        dx.start()
        dg.start()

    def two_chunks(cc, carry):
        for b in range(2):
            c = 2 * cc + b
            dx, dg = in_descs(c, b)
            dx.wait()
            dg.wait()

            @pl.when(c >= 2)
            def _wait_out():
                out_desc(c - 2, b).wait()

            bx, bg, bo = bufxs[b], bufgs[b], bufos[b]

            @plsc.parallel_loop(0, K, 1)
            def _add_row(r):
                for v in range(VECS_PER_ROW):
                    sl = pl.ds(v * LANES, LANES)
                    bo[r, sl] = bx[r, sl] + bg[r, sl]

            out_desc(c, b).start()

            @pl.when(c + 2 < CHUNKS)
            def _issue_next():
                ndx, ndg = in_descs(c + 2, b)
                ndx.start()
                ndg.start()
        return carry

    lax.fori_loop(0, CHUNKS // 2, two_chunks, 0, unroll=False)

    out_desc(CHUNKS - 2, 0).wait()
    out_desc(CHUNKS - 1, 1).wait()


@functools.partial(jax.jit, donate_argnums=())
def kernel(x, times_bt, embeddings_tc):
    xf = x.reshape(N, D)
    idx = times_bt.astype(jnp.int32).reshape(N)

    mesh = plsc.VectorSubcoreMesh(
        core_axis_name="c", subcore_axis_name="s", num_cores=NC,
        num_subcores=NS)
    run = pl.kernel(
        _sc_body,
        out_type=jax.ShapeDtypeStruct((N, D), jnp.float32),
        mesh=mesh,
        compiler_params=pltpu.CompilerParams(use_tc_tiling_on_sc=True),
        scratch_types=[
            pltpu.VMEM((ROWS_PER_W,), jnp.int32),
            pltpu.VMEM((K, D), jnp.float32),
            pltpu.VMEM((K, D), jnp.float32),
            pltpu.VMEM((K, D), jnp.float32),
            pltpu.VMEM((K, D), jnp.float32),
            pltpu.VMEM((K, D), jnp.float32),
            pltpu.VMEM((K, D), jnp.float32),
            pltpu.SemaphoreType.DMA,
            pltpu.SemaphoreType.DMA,
            pltpu.SemaphoreType.DMA,
            pltpu.SemaphoreType.DMA,
            pltpu.SemaphoreType.DMA,
            pltpu.SemaphoreType.DMA,
        ],
    )
    out = run(xf, idx, embeddings_tc)
    return out.reshape(NB, NT, D)
